# baseline (speedup 1.0000x reference)
import jax
import jax.numpy as jnp
from jax import lax
from jax.experimental import pallas as pl
from jax.experimental.pallas import tpu as pltpu
from jax.experimental.pallas import tpu_sc as plsc

B = 16384
U = 1000000
D = 64
H1 = 8
_INFO = plsc.get_sparse_core_info()
NC = _INFO.num_cores
NS = _INFO.num_subcores
L = _INFO.num_lanes
NW = NC * NS
BPW = B // NW
NBLK = BPW // L
TAIL0 = (U // 128) * 128
SMAX = TAIL0 - 128
CW = 65
B1_OFF = D * H1 * L
W2_OFF = B1_OFF + H1 * L
B2_OFF = W2_OFF + H1 * L
W_TOT = B2_OFF + L


def _sc_body(uidx_h, iidx_h, ut_h, it_h, tailu_h, taili_h, wb_h, out_h,
             uidx_v, iidx_v, ubuf, ibuf, ucmp, icmp, wb_v, out_v,
             sem0, sem1):
    wid = lax.axis_index("s") * NC + lax.axis_index("c")
    base = wid * BPW

    pltpu.sync_copy(uidx_h.at[pl.ds(base, BPW)], uidx_v)
    pltpu.sync_copy(iidx_h.at[pl.ds(base, BPW)], iidx_v)
    pltpu.sync_copy(wb_h, wb_v)
    pltpu.sync_copy(tailu_h, ubuf.at[2])
    pltpu.sync_copy(taili_h, ibuf.at[2])

    sems = (sem0, sem1)
    lanes = lax.iota(jnp.int32, L)
    dvecs = [16 * k + lanes for k in range(D // L)]

    def starts(vec):
        return jnp.minimum(
            lax.shift_left(lax.shift_right_logical(vec, 7), 7), SMAX)

    def fire(su, si, slot):
        su = pl.multiple_of(su, 128)
        si = pl.multiple_of(si, 128)
        sem = sems[slot]
        pltpu.async_copy(ut_h.at[:, pl.ds(su, 128)], ubuf.at[slot], sem)
        pltpu.async_copy(it_h.at[:, pl.ds(si, 128)], ibuf.at[slot], sem)

    def drain(slot):
        sem = sems[slot]
        pltpu.make_async_copy(ut_h.at[:, pl.ds(0, 128)], ubuf.at[slot],
                              sem).wait()
        pltpu.make_async_copy(it_h.at[:, pl.ds(0, 128)], ibuf.at[slot],
                              sem).wait()

    def extract(l16, rvu, rvi):
        slot = l16 & 1
        for (rv, buf, cmp) in ((rvu, ubuf, ucmp), (rvi, ibuf, icmp)):
            r = jnp.full((L,), rv[l16], dtype=jnp.int32)
            tail = r >= TAIL0
            offn = r & 127
            offt = jnp.maximum(r - TAIL0, 0)
            sv = jnp.full((L,), slot, dtype=jnp.int32)
            tv = jnp.full((L,), 2, dtype=jnp.int32)
            for k in range(D // L):
                vn = plsc.load_gather(buf, [sv, dvecs[k], offn])
                vt = plsc.load_gather(buf, [tv, dvecs[k], offt])
                cmp[pl.ds(l16 * CW + 16 * k, L)] = jnp.where(tail, vt, vn)

    ub = lanes * CW

    def mlp(blk):
        def d_body(d, accs):
            ucol = plsc.load_gather(ucmp, [ub + d])
            icol = plsc.load_gather(icmp, [ub + d])
            x = ucol * icol
            return tuple(
                accs[j] + x * wb_v[pl.ds((d * H1 + j) * L, L)]
                for j in range(H1))

        accs = lax.fori_loop(
            0, D, d_body,
            tuple(jnp.zeros((L,), jnp.float32) for _ in range(H1)),
            unroll=4)

        logit = wb_v[pl.ds(B2_OFF, L)]
        for j in range(H1):
            h = jnp.maximum(accs[j] + wb_v[pl.ds(B1_OFF + j * L, L)], 0.0)
            logit = logit + h * wb_v[pl.ds(W2_OFF + j * L, L)]
        sig = 1.0 / (1.0 + jnp.exp(-logit))
        out_v[pl.ds(blk * L, L)] = sig

    uv0 = uidx_v[pl.ds(0, L)]
    iv0 = iidx_v[pl.ds(0, L)]
    fire(starts(uv0)[0], starts(iv0)[0], 0)

    def blk_body(blk, carry):
        uv = uidx_v[pl.ds(blk * L, L)]
        iv = iidx_v[pl.ds(blk * L, L)]
        su = starts(uv)
        si = starts(iv)
        for l16 in range(L):
            if l16 + 1 < L:
                fire(su[l16 + 1], si[l16 + 1], (l16 + 1) & 1)
            else:
                def fire_next():
                    nuv = uidx_v[pl.ds((blk + 1) * L, L)]
                    niv = iidx_v[pl.ds((blk + 1) * L, L)]
                    fire(starts(nuv)[0], starts(niv)[0], 0)
                pl.when(blk + 1 < NBLK)(fire_next)
            drain(l16 & 1)
            extract(l16, uv, iv)
        mlp(blk)
        return carry

    lax.fori_loop(0, NBLK, blk_body, 0)
    pltpu.sync_copy(out_v, out_h.at[pl.ds(base, BPW)])


@jax.jit
def _run(uidx, iidx, utt, itt, tailu, taili, wb):
    mesh = plsc.VectorSubcoreMesh(core_axis_name="c", subcore_axis_name="s")
    f = pl.kernel(
        _sc_body,
        mesh=mesh,
        compiler_params=pltpu.CompilerParams(use_tc_tiling_on_sc=True,
                                             needs_layout_passes=False),
        out_type=jax.ShapeDtypeStruct((B,), jnp.float32),
        scratch_types=[
            pltpu.VMEM((BPW,), jnp.int32),
            pltpu.VMEM((BPW,), jnp.int32),
            pltpu.VMEM((3, D, 128), jnp.float32),
            pltpu.VMEM((3, D, 128), jnp.float32),
            pltpu.VMEM((L * CW,), jnp.float32),
            pltpu.VMEM((L * CW,), jnp.float32),
            pltpu.VMEM((W_TOT,), jnp.float32),
            pltpu.VMEM((BPW,), jnp.float32),
            pltpu.SemaphoreType.DMA,
            pltpu.SemaphoreType.DMA,
        ],
    )
    return f(uidx, iidx, utt, itt, tailu, taili, wb)


def kernel(group_inputs, user_inputs, item_inputs, user_table, item_table,
           W1, b1, W2, b2):
    del group_inputs
    uidx = user_inputs.astype(jnp.int32)
    iidx = item_inputs.astype(jnp.int32)
    utt = user_table.T
    itt = item_table.T
    zpad = jnp.zeros((D, 128 - (U - TAIL0)), dtype=jnp.float32)
    tailu = jnp.concatenate([user_table[TAIL0:].T, zpad], axis=1)
    taili = jnp.concatenate([item_table[TAIL0:].T, zpad], axis=1)
    w1b = jnp.broadcast_to(W1[:, :, None], (D, H1, L))
    b1b = jnp.broadcast_to(b1[:, None], (H1, L))
    w2b = jnp.broadcast_to(W2[:, 0][:, None], (H1, L))
    b2b = jnp.broadcast_to(b2, (L,))
    wb = jnp.concatenate([w1b.reshape(-1), b1b.reshape(-1),
                          w2b.reshape(-1), b2b]).astype(jnp.float32)
    out = _run(uidx, iidx, utt, itt, tailu, taili, wb)
    return out.reshape(B, 1)


# speedup vs baseline: 1.4915x; 1.4915x over previous
import jax
import jax.numpy as jnp
from jax import lax
from jax.experimental import pallas as pl
from jax.experimental.pallas import tpu as pltpu
from jax.experimental.pallas import tpu_sc as plsc

B = 16384
U = 1000000
D = 64
H1 = 8
_INFO = plsc.get_sparse_core_info()
NC = _INFO.num_cores
NS = _INFO.num_subcores
L = _INFO.num_lanes
NW = NC * NS
BPW = B // NW
NBLK = BPW // L
SLAB = 256
TAIL0 = (U // 128) * 128
SLABMAX = TAIL0 - SLAB
CW = 65
B1_OFF = D * H1 * L
W2_OFF = B1_OFF + H1 * L
B2_OFF = W2_OFF + H1 * L
W_TOT = B2_OFF + L


def _gather_body(uval_h, upos_h, ival_h, ipos_h, ut_h, it_h,
                 tailu_h, taili_h, su_h, si_h,
                 val_v, pos_v, posw, slb, tlb, cmp, sem0, sem1, ssem):
    wid = lax.axis_index("s") * NC + lax.axis_index("c")
    base = wid * BPW
    lanes = lax.iota(jnp.int32, L)
    dvecs = [16 * k + lanes for k in range(D // L)]
    sems = (sem0, sem1)

    for (valh, posh, tab, tailh, stg) in (
            (uval_h, upos_h, ut_h, tailu_h, su_h),
            (ival_h, ipos_h, it_h, taili_h, si_h)):
        pltpu.sync_copy(valh.at[pl.ds(base, BPW)], val_v.at[pl.ds(0, BPW)])
        pltpu.sync_copy(posh.at[pl.ds(base, BPW)], pos_v)
        pltpu.sync_copy(tailh, tlb)

        v0 = val_v[pl.ds(0, L)][0]
        base0 = jnp.minimum(
            lax.shift_left(lax.shift_right_logical(v0, 7), 7), SLABMAX)

        def sstart(k):
            return pl.multiple_of(
                jnp.minimum(base0 + k * SLAB, SLABMAX), 128)

        def fire(k, slot):
            pltpu.async_copy(tab.at[:, pl.ds(sstart(k), SLAB)],
                             slb.at[slot], sems[slot])

        def drain(slot):
            pltpu.make_async_copy(tab.at[:, pl.ds(0, SLAB)],
                                  slb.at[slot], sems[slot]).wait()

        def getval(p):
            sp = jnp.full((L,), p, dtype=jnp.int32)
            return plsc.load_gather(val_v, [sp])[0]

        def emit_row(p, offsp, srcbuf, pfx):
            rowsp = jnp.full((L,), p & 15, dtype=jnp.int32)
            for k in range(D // L):
                vals = plsc.load_gather(srcbuf, pfx + [dvecs[k], offsp])
                plsc.store_scatter(cmp, [rowsp, dvecs[k]], vals)

            def flush():
                blk0 = lax.shift_left(lax.shift_right_logical(p, 4), 4)
                posw[...] = pos_v[pl.ds(blk0, L)]
                pltpu.async_copy(cmp, stg.at[posw], ssem).wait()
            pl.when((p & 15) == 15)(flush)

        def consume(slot, k, p0):
            lo = sstart(k)
            hi = lo + SLAB
            slotsp = jnp.full((L,), slot, dtype=jnp.int32)

            def cond(p):
                return jnp.logical_and(p < BPW, getval(p) < hi)

            def body(p):
                offsp = jnp.broadcast_to(getval(p) - lo, (L,))
                emit_row(p, offsp, slb, [slotsp])
                return p + 1

            return lax.while_loop(cond, body, p0)

        fire(0, 0)
        fire(1, 1)

        def ocond(carry):
            p, k = carry
            return jnp.logical_and(p < BPW, base0 + k * SLAB < TAIL0)

        def obody(carry):
            p, k = carry
            drain(0)
            p = consume(0, k, p)
            fire(k + 2, 0)
            drain(1)
            p = consume(1, k + 1, p)
            fire(k + 3, 1)
            return (p, k + 2)

        p, _ = lax.while_loop(ocond, obody, (jnp.int32(0), jnp.int32(0)))
        drain(0)
        drain(1)

        def tcond(p):
            return p < BPW

        def tbody(p):
            offsp = jnp.broadcast_to(getval(p) - TAIL0, (L,))
            emit_row(p, offsp, tlb, [])
            return p + 1

        lax.while_loop(tcond, tbody, p)


def _mlp_body(su_h, si_h, wb_h, out_h,
              ub2, ib2, ucmp, icmp, wb_v, out_v):
    wid = lax.axis_index("s") * NC + lax.axis_index("c")
    base = wid * BPW
    lanes = lax.iota(jnp.int32, L)
    pltpu.sync_copy(wb_h, wb_v)
    ub = lanes * CW

    def blk_body(blk, carry):
        pltpu.sync_copy(su_h.at[pl.ds(base + blk * L, L)], ub2)
        pltpu.sync_copy(si_h.at[pl.ds(base + blk * L, L)], ib2)
        for l16 in range(L):
            for k in range(D // L):
                ucmp[pl.ds(l16 * CW + 16 * k, L)] = ub2[l16, pl.ds(16 * k, L)]
                icmp[pl.ds(l16 * CW + 16 * k, L)] = ib2[l16, pl.ds(16 * k, L)]

        def d_body(d, accs):
            ucol = plsc.load_gather(ucmp, [ub + d])
            icol = plsc.load_gather(icmp, [ub + d])
            x = ucol * icol
            return tuple(
                accs[j] + x * wb_v[pl.ds((d * H1 + j) * L, L)]
                for j in range(H1))

        accs = lax.fori_loop(
            0, D, d_body,
            tuple(jnp.zeros((L,), jnp.float32) for _ in range(H1)),
            unroll=4)

        logit = wb_v[pl.ds(B2_OFF, L)]
        for j in range(H1):
            h = jnp.maximum(accs[j] + wb_v[pl.ds(B1_OFF + j * L, L)], 0.0)
            logit = logit + h * wb_v[pl.ds(W2_OFF + j * L, L)]
        sig = 1.0 / (1.0 + jnp.exp(-logit))
        out_v[pl.ds(blk * L, L)] = sig
        return carry

    lax.fori_loop(0, NBLK, blk_body, 0)
    pltpu.sync_copy(out_v, out_h.at[pl.ds(base, BPW)])


@jax.jit
def _run(uval, upos, ival, ipos, utt, itt, tailu, taili, wb):
    mesh = plsc.VectorSubcoreMesh(core_axis_name="c", subcore_axis_name="s")
    cp = pltpu.CompilerParams(use_tc_tiling_on_sc=True,
                              needs_layout_passes=False)
    g = pl.kernel(
        _gather_body,
        mesh=mesh,
        compiler_params=cp,
        out_type=(jax.ShapeDtypeStruct((B, 128), jnp.float32),
                  jax.ShapeDtypeStruct((B, 128), jnp.float32)),
        scratch_types=[
            pltpu.VMEM((BPW + 2 * L,), jnp.int32),
            pltpu.VMEM((BPW,), jnp.int32),
            pltpu.VMEM((L,), jnp.int32),
            pltpu.VMEM((2, D, SLAB), jnp.float32),
            pltpu.VMEM((D, 128), jnp.float32),
            pltpu.VMEM((L, 128), jnp.float32),
            pltpu.SemaphoreType.DMA,
            pltpu.SemaphoreType.DMA,
            pltpu.SemaphoreType.DMA,
        ],
    )
    su, si = g(uval, upos, ival, ipos, utt, itt, tailu, taili)
    m = pl.kernel(
        _mlp_body,
        mesh=mesh,
        compiler_params=cp,
        out_type=jax.ShapeDtypeStruct((B,), jnp.float32),
        scratch_types=[
            pltpu.VMEM((L, 128), jnp.float32),
            pltpu.VMEM((L, 128), jnp.float32),
            pltpu.VMEM((L * CW,), jnp.float32),
            pltpu.VMEM((L * CW,), jnp.float32),
            pltpu.VMEM((W_TOT,), jnp.float32),
            pltpu.VMEM((BPW,), jnp.float32),
        ],
    )
    return m(su, si, wb)


def kernel(group_inputs, user_inputs, item_inputs, user_table, item_table,
           W1, b1, W2, b2):
    del group_inputs
    uidx = user_inputs.astype(jnp.int32)
    iidx = item_inputs.astype(jnp.int32)
    iota = lax.iota(jnp.int32, B)
    uval, upos = lax.sort_key_val(uidx, iota)
    ival, ipos = lax.sort_key_val(iidx, iota)
    utt = user_table.T
    itt = item_table.T
    zpad = jnp.zeros((D, 128 - (U - TAIL0)), dtype=jnp.float32)
    tailu = jnp.concatenate([user_table[TAIL0:].T, zpad], axis=1)
    taili = jnp.concatenate([item_table[TAIL0:].T, zpad], axis=1)
    w1b = jnp.broadcast_to(W1[:, :, None], (D, H1, L))
    b1b = jnp.broadcast_to(b1[:, None], (H1, L))
    w2b = jnp.broadcast_to(W2[:, 0][:, None], (H1, L))
    b2b = jnp.broadcast_to(b2, (L,))
    wb = jnp.concatenate([w1b.reshape(-1), b1b.reshape(-1),
                          w2b.reshape(-1), b2b]).astype(jnp.float32)
    out = _run(uval, upos, ival, ipos, utt, itt, tailu, taili, wb)
    return out.reshape(B, 1)


# speedup vs baseline: 1.7464x; 1.1709x over previous
import jax
import jax.numpy as jnp
from jax import lax
from jax.experimental import pallas as pl
from jax.experimental.pallas import tpu as pltpu
from jax.experimental.pallas import tpu_sc as plsc

B = 16384
U = 1000000
D = 64
H1 = 8
_INFO = plsc.get_sparse_core_info()
NC = _INFO.num_cores
NS = _INFO.num_subcores
L = _INFO.num_lanes
NW = NC * NS
BPW = B // NW
NBLK = BPW // L
SLAB = 512
TAIL0 = (U // 128) * 128
SLABMAX = TAIL0 - SLAB
CW = 65
B1_OFF = D * H1 * L
W2_OFF = B1_OFF + H1 * L
B2_OFF = W2_OFF + H1 * L
W_TOT = B2_OFF + L


def _gather_body(uval_h, upos_h, ival_h, ipos_h, ut_h, it_h,
                 tailu_h, taili_h, su_h, si_h,
                 val_v, pos_v, posw, slb, tlb, cmp, sem0, sem1, ssem):
    wid = lax.axis_index("s") * NC + lax.axis_index("c")
    base = wid * BPW
    lanes = lax.iota(jnp.int32, L)
    dvecs = [16 * k + lanes for k in range(D // L)]
    sems = (sem0, sem1)

    for (valh, posh, tab, tailh, stg) in (
            (uval_h, upos_h, ut_h, tailu_h, su_h),
            (ival_h, ipos_h, it_h, taili_h, si_h)):
        pltpu.sync_copy(valh.at[pl.ds(base, BPW)], val_v.at[pl.ds(0, BPW)])
        pltpu.sync_copy(posh.at[pl.ds(base, BPW)], pos_v)
        pltpu.sync_copy(tailh, tlb)

        v0 = val_v[pl.ds(0, L)][0]
        base0 = jnp.minimum(
            lax.shift_left(lax.shift_right_logical(v0, 7), 7), SLABMAX)

        def sstart(k):
            return pl.multiple_of(
                jnp.minimum(base0 + k * SLAB, SLABMAX), 128)

        def fire(k, slot):
            pltpu.async_copy(tab.at[:, pl.ds(sstart(k), SLAB)],
                             slb.at[slot], sems[slot])

        def drain(slot):
            pltpu.make_async_copy(tab.at[:, pl.ds(0, SLAB)],
                                  slb.at[slot], sems[slot]).wait()

        def getval(p):
            sp = jnp.full((L,), p, dtype=jnp.int32)
            return plsc.load_gather(val_v, [sp])[0]

        def emit_row(p, offsp, srcbuf, pfx):
            rowsp = jnp.full((L,), p & 15, dtype=jnp.int32)
            for k in range(D // L):
                vals = plsc.load_gather(srcbuf, pfx + [dvecs[k], offsp])
                plsc.store_scatter(cmp, [rowsp, dvecs[k]], vals)

            def flush():
                blk0 = lax.shift_left(lax.shift_right_logical(p, 4), 4)
                posw[...] = pos_v[pl.ds(blk0, L)]
                pltpu.async_copy(cmp, stg.at[posw], ssem).wait()
            pl.when((p & 15) == 15)(flush)

        def consume(slot, k, p0):
            lo = sstart(k)
            hi = lo + SLAB
            slotsp = jnp.full((L,), slot, dtype=jnp.int32)

            def cond(p):
                return jnp.logical_and(p < BPW, getval(p) < hi)

            def body(p):
                offsp = jnp.broadcast_to(getval(p) - lo, (L,))
                emit_row(p, offsp, slb, [slotsp])
                return p + 1

            return lax.while_loop(cond, body, p0)

        fire(0, 0)
        fire(1, 1)

        def ocond(carry):
            p, k = carry
            return jnp.logical_and(p < BPW, base0 + k * SLAB < TAIL0)

        def obody(carry):
            p, k = carry
            drain(0)
            p = consume(0, k, p)
            fire(k + 2, 0)
            drain(1)
            p = consume(1, k + 1, p)
            fire(k + 3, 1)
            return (p, k + 2)

        p, _ = lax.while_loop(ocond, obody, (jnp.int32(0), jnp.int32(0)))
        drain(0)
        drain(1)

        def tcond(p):
            return p < BPW

        def tbody(p):
            offsp = jnp.broadcast_to(getval(p) - TAIL0, (L,))
            emit_row(p, offsp, tlb, [])
            return p + 1

        lax.while_loop(tcond, tbody, p)


def _mlp_body(su_h, si_h, wb_h, out_h,
              ub2, ib2, ucmp, icmp, wb_v, out_v):
    wid = lax.axis_index("s") * NC + lax.axis_index("c")
    base = wid * BPW
    lanes = lax.iota(jnp.int32, L)
    pltpu.sync_copy(wb_h, wb_v)
    ub = lanes * CW

    def sb_body(sb, carry):
        pltpu.sync_copy(su_h.at[pl.ds(base + sb * 8 * L, 8 * L)], ub2)
        pltpu.sync_copy(si_h.at[pl.ds(base + sb * 8 * L, 8 * L)], ib2)
        for bi in range(8):
            blk_body(sb * 8 + bi, bi)
        return carry

    def blk_body(blk, bi):
        for l16 in range(L):
            row = bi * L + l16
            for k in range(D // L):
                ucmp[pl.ds(l16 * CW + 16 * k, L)] = ub2[row, pl.ds(16 * k, L)]
                icmp[pl.ds(l16 * CW + 16 * k, L)] = ib2[row, pl.ds(16 * k, L)]

        def d_body(d, accs):
            ucol = plsc.load_gather(ucmp, [ub + d])
            icol = plsc.load_gather(icmp, [ub + d])
            x = ucol * icol
            return tuple(
                accs[j] + x * wb_v[pl.ds((d * H1 + j) * L, L)]
                for j in range(H1))

        accs = lax.fori_loop(
            0, D, d_body,
            tuple(jnp.zeros((L,), jnp.float32) for _ in range(H1)),
            unroll=4)

        logit = wb_v[pl.ds(B2_OFF, L)]
        for j in range(H1):
            h = jnp.maximum(accs[j] + wb_v[pl.ds(B1_OFF + j * L, L)], 0.0)
            logit = logit + h * wb_v[pl.ds(W2_OFF + j * L, L)]
        sig = 1.0 / (1.0 + jnp.exp(-logit))
        out_v[pl.ds(blk * L, L)] = sig

    lax.fori_loop(0, NBLK // 8, sb_body, 0)
    pltpu.sync_copy(out_v, out_h.at[pl.ds(base, BPW)])


@jax.jit
def _run(uval, upos, ival, ipos, utt, itt, tailu, taili, wb):
    mesh = plsc.VectorSubcoreMesh(core_axis_name="c", subcore_axis_name="s")
    cp = pltpu.CompilerParams(use_tc_tiling_on_sc=True,
                              needs_layout_passes=False)
    g = pl.kernel(
        _gather_body,
        mesh=mesh,
        compiler_params=cp,
        out_type=(jax.ShapeDtypeStruct((B, 128), jnp.float32),
                  jax.ShapeDtypeStruct((B, 128), jnp.float32)),
        scratch_types=[
            pltpu.VMEM((BPW + 2 * L,), jnp.int32),
            pltpu.VMEM((BPW,), jnp.int32),
            pltpu.VMEM((L,), jnp.int32),
            pltpu.VMEM((2, D, SLAB), jnp.float32),
            pltpu.VMEM((D, 128), jnp.float32),
            pltpu.VMEM((L, 128), jnp.float32),
            pltpu.SemaphoreType.DMA,
            pltpu.SemaphoreType.DMA,
            pltpu.SemaphoreType.DMA,
        ],
    )
    su, si = g(uval, upos, ival, ipos, utt, itt, tailu, taili)
    m = pl.kernel(
        _mlp_body,
        mesh=mesh,
        compiler_params=cp,
        out_type=jax.ShapeDtypeStruct((B,), jnp.float32),
        scratch_types=[
            pltpu.VMEM((8 * L, 128), jnp.float32),
            pltpu.VMEM((8 * L, 128), jnp.float32),
            pltpu.VMEM((L * CW,), jnp.float32),
            pltpu.VMEM((L * CW,), jnp.float32),
            pltpu.VMEM((W_TOT,), jnp.float32),
            pltpu.VMEM((BPW,), jnp.float32),
        ],
    )
    return m(su, si, wb)


def kernel(group_inputs, user_inputs, item_inputs, user_table, item_table,
           W1, b1, W2, b2):
    del group_inputs
    uidx = user_inputs.astype(jnp.int32)
    iidx = item_inputs.astype(jnp.int32)
    iota = lax.iota(jnp.int32, B)
    uval, upos = lax.sort_key_val(uidx, iota)
    ival, ipos = lax.sort_key_val(iidx, iota)
    utt = user_table.T
    itt = item_table.T
    zpad = jnp.zeros((D, 128 - (U - TAIL0)), dtype=jnp.float32)
    tailu = jnp.concatenate([user_table[TAIL0:].T, zpad], axis=1)
    taili = jnp.concatenate([item_table[TAIL0:].T, zpad], axis=1)
    w1b = jnp.broadcast_to(W1[:, :, None], (D, H1, L))
    b1b = jnp.broadcast_to(b1[:, None], (H1, L))
    w2b = jnp.broadcast_to(W2[:, 0][:, None], (H1, L))
    b2b = jnp.broadcast_to(b2, (L,))
    wb = jnp.concatenate([w1b.reshape(-1), b1b.reshape(-1),
                          w2b.reshape(-1), b2b]).astype(jnp.float32)
    out = _run(uval, upos, ival, ipos, utt, itt, tailu, taili, wb)
    return out.reshape(B, 1)


# speedup vs baseline: 1.7567x; 1.0059x over previous
import jax
import jax.numpy as jnp
from jax import lax
from jax.experimental import pallas as pl
from jax.experimental.pallas import tpu as pltpu
from jax.experimental.pallas import tpu_sc as plsc

B = 16384
U = 1000000
D = 64
H1 = 8
_INFO = plsc.get_sparse_core_info()
NC = _INFO.num_cores
NS = _INFO.num_subcores
L = _INFO.num_lanes
NW = NC * NS
BPW = B // NW
NBLK = BPW // L
SLAB = 768
TAIL0 = (U // 128) * 128
SLABMAX = TAIL0 - SLAB
CW = 65
B1_OFF = D * H1 * L
W2_OFF = B1_OFF + H1 * L
B2_OFF = W2_OFF + H1 * L
W_TOT = B2_OFF + L


def _gather_body(uval_h, upos_h, ival_h, ipos_h, ut_h, it_h,
                 tailu_h, taili_h, su_h, si_h,
                 val_v, pos_v, posw, slb, tlb, cmp, sem0, sem1, ssem):
    wid = lax.axis_index("s") * NC + lax.axis_index("c")
    base = wid * BPW
    lanes = lax.iota(jnp.int32, L)
    dvecs = [16 * k + lanes for k in range(D // L)]
    sems = (sem0, sem1)

    for (valh, posh, tab, tailh, stg) in (
            (uval_h, upos_h, ut_h, tailu_h, su_h),
            (ival_h, ipos_h, it_h, taili_h, si_h)):
        pltpu.sync_copy(valh.at[pl.ds(base, BPW)], val_v.at[pl.ds(0, BPW)])
        pltpu.sync_copy(posh.at[pl.ds(base, BPW)], pos_v)
        pltpu.sync_copy(tailh, tlb)

        v0 = val_v[pl.ds(0, L)][0]
        base0 = jnp.minimum(
            lax.shift_left(lax.shift_right_logical(v0, 7), 7), SLABMAX)

        def sstart(k):
            return pl.multiple_of(
                jnp.minimum(base0 + k * SLAB, SLABMAX), 128)

        def fire(k, slot):
            pltpu.async_copy(tab.at[:, pl.ds(sstart(k), SLAB)],
                             slb.at[slot], sems[slot])

        def drain(slot):
            pltpu.make_async_copy(tab.at[:, pl.ds(0, SLAB)],
                                  slb.at[slot], sems[slot]).wait()

        def getval(p):
            sp = jnp.full((L,), p, dtype=jnp.int32)
            return plsc.load_gather(val_v, [sp])[0]

        def emit_row(p, offsp, srcbuf, pfx):
            rowsp = jnp.full((L,), p & 15, dtype=jnp.int32)
            for k in range(D // L):
                vals = plsc.load_gather(srcbuf, pfx + [dvecs[k], offsp])
                plsc.store_scatter(cmp, [rowsp, dvecs[k]], vals)

            def flush():
                blk0 = lax.shift_left(lax.shift_right_logical(p, 4), 4)
                posw[...] = pos_v[pl.ds(blk0, L)]
                pltpu.async_copy(cmp, stg.at[posw], ssem).wait()
            pl.when((p & 15) == 15)(flush)

        def consume(slot, k, p0):
            lo = sstart(k)
            hi = lo + SLAB
            slotsp = jnp.full((L,), slot, dtype=jnp.int32)

            def cond(p):
                return jnp.logical_and(p < BPW, getval(p) < hi)

            def body(p):
                offsp = jnp.broadcast_to(getval(p) - lo, (L,))
                emit_row(p, offsp, slb, [slotsp])
                return p + 1

            return lax.while_loop(cond, body, p0)

        fire(0, 0)
        fire(1, 1)

        def ocond(carry):
            p, k = carry
            return jnp.logical_and(p < BPW, base0 + k * SLAB < TAIL0)

        def obody(carry):
            p, k = carry
            drain(0)
            p = consume(0, k, p)
            fire(k + 2, 0)
            drain(1)
            p = consume(1, k + 1, p)
            fire(k + 3, 1)
            return (p, k + 2)

        p, _ = lax.while_loop(ocond, obody, (jnp.int32(0), jnp.int32(0)))
        drain(0)
        drain(1)

        def tcond(p):
            return p < BPW

        def tbody(p):
            offsp = jnp.broadcast_to(getval(p) - TAIL0, (L,))
            emit_row(p, offsp, tlb, [])
            return p + 1

        lax.while_loop(tcond, tbody, p)


def _mlp_body(su_h, si_h, wb_h, out_h,
              ub2, ib2, ucmp, icmp, wb_v, out_v):
    wid = lax.axis_index("s") * NC + lax.axis_index("c")
    base = wid * BPW
    lanes = lax.iota(jnp.int32, L)
    pltpu.sync_copy(wb_h, wb_v)
    ub = lanes * CW

    def sb_body(sb, carry):
        pltpu.sync_copy(su_h.at[pl.ds(base + sb * 8 * L, 8 * L)], ub2)
        pltpu.sync_copy(si_h.at[pl.ds(base + sb * 8 * L, 8 * L)], ib2)
        for bi in range(8):
            blk_body(sb * 8 + bi, bi)
        return carry

    def blk_body(blk, bi):
        for l16 in range(L):
            row = bi * L + l16
            for k in range(D // L):
                ucmp[pl.ds(l16 * CW + 16 * k, L)] = ub2[row, pl.ds(16 * k, L)]
                icmp[pl.ds(l16 * CW + 16 * k, L)] = ib2[row, pl.ds(16 * k, L)]

        def d_body(d, accs):
            ucol = plsc.load_gather(ucmp, [ub + d])
            icol = plsc.load_gather(icmp, [ub + d])
            x = ucol * icol
            return tuple(
                accs[j] + x * wb_v[pl.ds((d * H1 + j) * L, L)]
                for j in range(H1))

        accs = lax.fori_loop(
            0, D, d_body,
            tuple(jnp.zeros((L,), jnp.float32) for _ in range(H1)),
            unroll=4)

        logit = wb_v[pl.ds(B2_OFF, L)]
        for j in range(H1):
            h = jnp.maximum(accs[j] + wb_v[pl.ds(B1_OFF + j * L, L)], 0.0)
            logit = logit + h * wb_v[pl.ds(W2_OFF + j * L, L)]
        sig = 1.0 / (1.0 + jnp.exp(-logit))
        out_v[pl.ds(blk * L, L)] = sig

    lax.fori_loop(0, NBLK // 8, sb_body, 0)
    pltpu.sync_copy(out_v, out_h.at[pl.ds(base, BPW)])


@jax.jit
def _run(uval, upos, ival, ipos, utt, itt, tailu, taili, wb):
    mesh = plsc.VectorSubcoreMesh(core_axis_name="c", subcore_axis_name="s")
    cp = pltpu.CompilerParams(use_tc_tiling_on_sc=True,
                              needs_layout_passes=False)
    g = pl.kernel(
        _gather_body,
        mesh=mesh,
        compiler_params=cp,
        out_type=(jax.ShapeDtypeStruct((B, 128), jnp.float32),
                  jax.ShapeDtypeStruct((B, 128), jnp.float32)),
        scratch_types=[
            pltpu.VMEM((BPW + 2 * L,), jnp.int32),
            pltpu.VMEM((BPW,), jnp.int32),
            pltpu.VMEM((L,), jnp.int32),
            pltpu.VMEM((2, D, SLAB), jnp.float32),
            pltpu.VMEM((D, 128), jnp.float32),
            pltpu.VMEM((L, 128), jnp.float32),
            pltpu.SemaphoreType.DMA,
            pltpu.SemaphoreType.DMA,
            pltpu.SemaphoreType.DMA,
        ],
    )
    su, si = g(uval, upos, ival, ipos, utt, itt, tailu, taili)
    m = pl.kernel(
        _mlp_body,
        mesh=mesh,
        compiler_params=cp,
        out_type=jax.ShapeDtypeStruct((B,), jnp.float32),
        scratch_types=[
            pltpu.VMEM((8 * L, 128), jnp.float32),
            pltpu.VMEM((8 * L, 128), jnp.float32),
            pltpu.VMEM((L * CW,), jnp.float32),
            pltpu.VMEM((L * CW,), jnp.float32),
            pltpu.VMEM((W_TOT,), jnp.float32),
            pltpu.VMEM((BPW,), jnp.float32),
        ],
    )
    return m(su, si, wb)


def kernel(group_inputs, user_inputs, item_inputs, user_table, item_table,
           W1, b1, W2, b2):
    del group_inputs
    uidx = user_inputs.astype(jnp.int32)
    iidx = item_inputs.astype(jnp.int32)
    iota = lax.iota(jnp.int32, B)
    uval, upos = lax.sort_key_val(uidx, iota)
    ival, ipos = lax.sort_key_val(iidx, iota)
    utt = user_table.T
    itt = item_table.T
    zpad = jnp.zeros((D, 128 - (U - TAIL0)), dtype=jnp.float32)
    tailu = jnp.concatenate([user_table[TAIL0:].T, zpad], axis=1)
    taili = jnp.concatenate([item_table[TAIL0:].T, zpad], axis=1)
    w1b = jnp.broadcast_to(W1[:, :, None], (D, H1, L))
    b1b = jnp.broadcast_to(b1[:, None], (H1, L))
    w2b = jnp.broadcast_to(W2[:, 0][:, None], (H1, L))
    b2b = jnp.broadcast_to(b2, (L,))
    wb = jnp.concatenate([w1b.reshape(-1), b1b.reshape(-1),
                          w2b.reshape(-1), b2b]).astype(jnp.float32)
    out = _run(uval, upos, ival, ipos, utt, itt, tailu, taili, wb)
    return out.reshape(B, 1)


# speedup vs baseline: 1.7792x; 1.0128x over previous
import jax
import jax.numpy as jnp
from jax import lax
from jax.experimental import pallas as pl
from jax.experimental.pallas import tpu as pltpu
from jax.experimental.pallas import tpu_sc as plsc

B = 16384
U = 1000000
D = 64
H1 = 8
_INFO = plsc.get_sparse_core_info()
NC = _INFO.num_cores
NS = _INFO.num_subcores
L = _INFO.num_lanes
NW = NC * NS
BPW = B // NW
NBLK = BPW // L
SLAB = 768
TAIL0 = (U // 128) * 128
SLABMAX = TAIL0 - SLAB
CW = 65
B1_OFF = D * H1 * L
W2_OFF = B1_OFF + H1 * L
B2_OFF = W2_OFF + H1 * L
W_TOT = B2_OFF + L


def _gather_body(uval_h, upos_h, ival_h, ipos_h, ut_h, it_h,
                 tailu_h, taili_h, su_h, si_h,
                 val_v, pos_v, posw, slb, tlb, cmp, sem0, sem1, ssem):
    wid = lax.axis_index("s") * NC + lax.axis_index("c")
    base = wid * BPW
    lanes = lax.iota(jnp.int32, L)
    dvecs = [16 * k + lanes for k in range(D // L)]
    sems = (sem0, sem1)

    for (valh, posh, tab, tailh, stg) in (
            (uval_h, upos_h, ut_h, tailu_h, su_h),
            (ival_h, ipos_h, it_h, taili_h, si_h)):
        pltpu.sync_copy(valh.at[pl.ds(base, BPW)], val_v.at[pl.ds(0, BPW)])
        pltpu.sync_copy(posh.at[pl.ds(base, BPW)], pos_v)
        pltpu.sync_copy(tailh, tlb)

        v0 = val_v[pl.ds(0, L)][0]
        base0 = jnp.minimum(
            lax.shift_left(lax.shift_right_logical(v0, 7), 7), SLABMAX)

        def sstart(k):
            return pl.multiple_of(
                jnp.minimum(base0 + k * SLAB, SLABMAX), 128)

        def fire(k, slot):
            pltpu.async_copy(tab.at[:, pl.ds(sstart(k), SLAB)],
                             slb.at[slot], sems[slot])

        def drain(slot):
            pltpu.make_async_copy(tab.at[:, pl.ds(0, SLAB)],
                                  slb.at[slot], sems[slot]).wait()

        def getval(p):
            sp = jnp.full((L,), p, dtype=jnp.int32)
            return plsc.load_gather(val_v, [sp])[0]

        def emit_row(p, offsp, srcbuf, pfx):
            rowsp = jnp.full((L,), p & 15, dtype=jnp.int32)
            for k in range(D // L):
                vals = plsc.load_gather(srcbuf, pfx + [dvecs[k], offsp])
                plsc.store_scatter(cmp, [rowsp, dvecs[k]], vals)

            def flush():
                blk0 = lax.shift_left(lax.shift_right_logical(p, 4), 4)
                posw[...] = pos_v[pl.ds(blk0, L)]
                pltpu.async_copy(cmp, stg.at[posw], ssem).wait()
            pl.when((p & 15) == 15)(flush)

        def consume(slot, k, p0):
            lo = sstart(k)
            hi = lo + SLAB
            slotsp = jnp.full((L,), slot, dtype=jnp.int32)

            def cond(p):
                return jnp.logical_and(p < BPW, getval(p) < hi)

            def body(p):
                offsp = jnp.broadcast_to(getval(p) - lo, (L,))
                emit_row(p, offsp, slb, [slotsp])
                return p + 1

            return lax.while_loop(cond, body, p0)

        fire(0, 0)
        fire(1, 1)

        def ocond(carry):
            p, k = carry
            return jnp.logical_and(p < BPW, base0 + k * SLAB < TAIL0)

        def obody(carry):
            p, k = carry
            drain(0)
            p = consume(0, k, p)
            fire(k + 2, 0)
            drain(1)
            p = consume(1, k + 1, p)
            fire(k + 3, 1)
            return (p, k + 2)

        p, _ = lax.while_loop(ocond, obody, (jnp.int32(0), jnp.int32(0)))
        drain(0)
        drain(1)

        def tcond(p):
            return p < BPW

        def tbody(p):
            offsp = jnp.broadcast_to(getval(p) - TAIL0, (L,))
            emit_row(p, offsp, tlb, [])
            return p + 1

        lax.while_loop(tcond, tbody, p)


def _mlp_body(su_h, si_h, wb_h, out_h,
              ub2, ib2, ucmp, icmp, wb_v, out_v):
    wid = lax.axis_index("s") * NC + lax.axis_index("c")
    base = wid * BPW
    lanes = lax.iota(jnp.int32, L)
    pltpu.sync_copy(wb_h, wb_v)
    ub = lanes * CW

    def sb_body(sb, carry):
        pltpu.sync_copy(su_h.at[pl.ds(base + sb * 8 * L, 8 * L)], ub2)
        pltpu.sync_copy(si_h.at[pl.ds(base + sb * 8 * L, 8 * L)], ib2)
        for bi in range(8):
            blk_body(sb * 8 + bi, bi)
        return carry

    def blk_body(blk, bi):
        for l16 in range(L):
            row = bi * L + l16
            for k in range(D // L):
                ucmp[pl.ds(l16 * CW + 16 * k, L)] = ub2[row, pl.ds(16 * k, L)]
                icmp[pl.ds(l16 * CW + 16 * k, L)] = ib2[row, pl.ds(16 * k, L)]

        def d_body(d, accs):
            ucol = plsc.load_gather(ucmp, [ub + d])
            icol = plsc.load_gather(icmp, [ub + d])
            x = ucol * icol
            return tuple(
                accs[j] + x * wb_v[pl.ds((d * H1 + j) * L, L)]
                for j in range(H1))

        accs = lax.fori_loop(
            0, D, d_body,
            tuple(jnp.zeros((L,), jnp.float32) for _ in range(H1)),
            unroll=4)

        logit = wb_v[pl.ds(B2_OFF, L)]
        for j in range(H1):
            h = jnp.maximum(accs[j] + wb_v[pl.ds(B1_OFF + j * L, L)], 0.0)
            logit = logit + h * wb_v[pl.ds(W2_OFF + j * L, L)]
        sig = 1.0 / (1.0 + jnp.exp(-logit))
        out_v[pl.ds(blk * L, L)] = sig

    lax.fori_loop(0, NBLK // 8, sb_body, 0)
    pltpu.sync_copy(out_v, out_h.at[pl.ds(base, BPW)])


@jax.jit
def _run(uval, upos, ival, ipos, utt, itt, tailu, taili, wb):
    mesh = plsc.VectorSubcoreMesh(core_axis_name="c", subcore_axis_name="s")
    cp = pltpu.CompilerParams(use_tc_tiling_on_sc=True,
                              needs_layout_passes=False)
    g = pl.kernel(
        _gather_body,
        mesh=mesh,
        compiler_params=cp,
        out_type=(jax.ShapeDtypeStruct((B, 128), jnp.float32),
                  jax.ShapeDtypeStruct((B, 128), jnp.float32)),
        scratch_types=[
            pltpu.VMEM((BPW + 2 * L,), jnp.int32),
            pltpu.VMEM((BPW,), jnp.int32),
            pltpu.VMEM((L,), jnp.int32),
            pltpu.VMEM((2, D, SLAB), jnp.float32),
            pltpu.VMEM((D, 128), jnp.float32),
            pltpu.VMEM((L, 128), jnp.float32),
            pltpu.SemaphoreType.DMA,
            pltpu.SemaphoreType.DMA,
            pltpu.SemaphoreType.DMA,
        ],
    )
    su, si = g(uval, upos, ival, ipos, utt, itt, tailu, taili)
    m = pl.kernel(
        _mlp_body,
        mesh=mesh,
        compiler_params=cp,
        out_type=jax.ShapeDtypeStruct((B,), jnp.float32),
        scratch_types=[
            pltpu.VMEM((8 * L, 128), jnp.float32),
            pltpu.VMEM((8 * L, 128), jnp.float32),
            pltpu.VMEM((L * CW,), jnp.float32),
            pltpu.VMEM((L * CW,), jnp.float32),
            pltpu.VMEM((W_TOT,), jnp.float32),
            pltpu.VMEM((BPW,), jnp.float32),
        ],
    )
    return m(su, si, wb)


def kernel(group_inputs, user_inputs, item_inputs, user_table, item_table,
           W1, b1, W2, b2):
    del group_inputs
    uidx = user_inputs.astype(jnp.int32)
    iidx = item_inputs.astype(jnp.int32)
    keys = jnp.concatenate([uidx, iidx + (1 << 20)])
    skeys, spos = lax.sort_key_val(keys, lax.iota(jnp.int32, 2 * B))
    uval, upos = skeys[:B], spos[:B]
    ival, ipos = skeys[B:] - (1 << 20), spos[B:] - B
    utt = user_table.T
    itt = item_table.T
    zpad = jnp.zeros((D, 128 - (U - TAIL0)), dtype=jnp.float32)
    tailu = jnp.concatenate([user_table[TAIL0:].T, zpad], axis=1)
    taili = jnp.concatenate([item_table[TAIL0:].T, zpad], axis=1)
    w1b = jnp.broadcast_to(W1[:, :, None], (D, H1, L))
    b1b = jnp.broadcast_to(b1[:, None], (H1, L))
    w2b = jnp.broadcast_to(W2[:, 0][:, None], (H1, L))
    b2b = jnp.broadcast_to(b2, (L,))
    wb = jnp.concatenate([w1b.reshape(-1), b1b.reshape(-1),
                          w2b.reshape(-1), b2b]).astype(jnp.float32)
    out = _run(uval, upos, ival, ipos, utt, itt, tailu, taili, wb)
    return out.reshape(B, 1)
